# baseline (speedup 1.0000x reference)
import functools

import jax
import jax.numpy as jnp
from jax import lax
from jax.experimental import pallas as pl
from jax.experimental.pallas import tpu as pltpu
from jax.experimental.pallas import tpu_sc as plsc

N = 10000
E = 320000
D = 128
H = 64
S = 64
L = 256

NC = 2
NS = 16
NW = NC * NS
EW = E // NW
C = 80
NCH = EW // C
CM = 40
NCHM = EW // CM
NP = 10240
RT = NP // NS

_mesh = plsc.VectorSubcoreMesh(core_axis_name="c", subcore_axis_name="s")
_sc_params = pltpu.CompilerParams(use_tc_tiling_on_sc=False)

f32 = jnp.float32
i32 = jnp.int32


@functools.partial(
    pl.kernel,
    out_type=jax.ShapeDtypeStruct((NC, NP, H), f32),
    mesh=_mesh,
    compiler_params=_sc_params,
    scratch_types=[
        pltpu.VMEM((EW,), i32),
        pltpu.VMEM((C, 16), f32),
        pltpu.VMEM((RT, 16), f32),
        pltpu.VMEM((RT, H), f32),
        pltpu.VMEM_SHARED((NP, 16), f32),
        pltpu.SemaphoreType.DMA,
    ],
)
def _deg_kernel(dst_hbm, zeros16_hbm, out_hbm, idx_stage, ones_c,
                cnt16, cnt64, acc, sem):
    c = lax.axis_index("c")
    s = lax.axis_index("s")
    wid = c * NS + s

    pltpu.sync_copy(zeros16_hbm.at[pl.ds(s * RT, RT)],
                    acc.at[pl.ds(s * RT, RT)])

    def fill(i, carry):
        ones_c[i, :] = jnp.full((16,), 1.0, f32)
        return carry

    lax.fori_loop(0, C, fill, 0)
    pltpu.sync_copy(dst_hbm.at[pl.ds(wid * EW, EW)], idx_stage)
    plsc.subcore_barrier()

    def chunk(j, carry):
        pltpu.async_copy(ones_c, acc.at[idx_stage.at[pl.ds(j * C, C)]],
                         sem, add=True)
        return carry

    lax.fori_loop(0, NCH, chunk, 0)

    def drain(j, carry):
        pltpu.make_async_copy(ones_c, acc.at[idx_stage.at[pl.ds(0, C)]],
                              sem).wait()
        return carry

    lax.fori_loop(0, NCH, drain, 0)
    plsc.subcore_barrier()

    pltpu.sync_copy(acc.at[pl.ds(s * RT, RT)], cnt16)

    def expand(r, carry):
        v = cnt16[r, :]
        for k in range(H // 16):
            cnt64[r, pl.ds(16 * k, 16)] = v
        return carry

    lax.fori_loop(0, RT, expand, 0)
    pltpu.sync_copy(cnt64, out_hbm.at[c, pl.ds(s * RT, RT)])


@functools.partial(
    pl.kernel,
    out_type=jax.ShapeDtypeStruct((NC, NP, H), f32),
    mesh=_mesh,
    compiler_params=_sc_params,
    scratch_types=[
        pltpu.VMEM((EW,), i32),
        pltpu.VMEM((EW,), i32),
        pltpu.VMEM((4 * 5, CM, H), f32),
        pltpu.VMEM_SHARED((NP, H), f32),
        pltpu.SemaphoreType.DMA,
        pltpu.SemaphoreType.DMA,
        pltpu.SemaphoreType.DMA,
        pltpu.SemaphoreType.DMA,
        pltpu.SemaphoreType.DMA,
        pltpu.SemaphoreType.DMA,
        pltpu.SemaphoreType.DMA,
        pltpu.SemaphoreType.DMA,
    ],
)
def _mp_kernel(ps_hbm, src_hbm, dst_hbm, zeros_hbm, out_hbm,
               src_stage, dst_stage, bufs, acc,
               gsem0, gsem1, gsem2, gsem3, ssem0, ssem1, ssem2, ssem3):
    c = lax.axis_index("c")
    s = lax.axis_index("s")
    wid = c * NS + s
    gsems = (gsem0, gsem1, gsem2, gsem3)
    ssems = (ssem0, ssem1, ssem2, ssem3)
    NSET = 4
    R = 5
    NG = NCHM // R

    pltpu.sync_copy(zeros_hbm.at[pl.ds(s * RT, RT)], acc.at[pl.ds(s * RT, RT)])
    pltpu.sync_copy(src_hbm.at[pl.ds(wid * EW, EW)], src_stage)
    pltpu.sync_copy(dst_hbm.at[pl.ds(wid * EW, EW)], dst_stage)
    plsc.subcore_barrier()

    def gather(j, p, b):
        pltpu.async_copy(ps_hbm.at[src_stage.at[pl.ds(j * CM, CM)]],
                         bufs.at[p * R + b], gsems[p])

    def drain_scatters(p):
        for b in range(R):
            pltpu.make_async_copy(bufs.at[p * R + b],
                                  acc.at[dst_stage.at[pl.ds(0, CM)]],
                                  ssems[p]).wait()

    def slot(g, p, drain_prev=True, refill=True):
        for b in range(R):
            pltpu.make_async_copy(ps_hbm.at[pl.ds(0, CM)],
                                  bufs.at[p * R + b], gsems[p]).wait()
        for b in range(R):
            j = g * R + b
            pltpu.async_copy(bufs.at[p * R + b],
                             acc.at[dst_stage.at[pl.ds(j * CM, CM)]],
                             ssems[p], add=True)
        p2 = (p + 2) % NSET
        if drain_prev:
            @pl.when(g > 1)
            def _():
                drain_scatters(p2)
        if refill:
            @pl.when(g + 2 < NG)
            def _():
                for b in range(R):
                    gather((g + 2) * R + b, p2, b)

    for p in (0, 1):
        for b in range(R):
            gather(p * R + b, p, b)

    def outer(gg, carry):
        for p in (0, 1, 2, 3):
            slot(4 * gg + p, p)
        return carry

    lax.fori_loop(0, (NG - 2) // 4, outer, 0)
    slot(NG - 2, (NG - 2) % NSET, refill=False)
    slot(NG - 1, (NG - 1) % NSET, refill=False)
    drain_scatters((NG - 2) % NSET)
    drain_scatters((NG - 1) % NSET)

    plsc.subcore_barrier()
    pltpu.sync_copy(acc.at[pl.ds(s * RT, RT)], out_hbm.at[c, pl.ds(s * RT, RT)])


SIDX_W = S * L // NW
GCH = 128


@functools.partial(
    pl.kernel,
    out_type=jax.ShapeDtypeStruct((NW, 2, H), f32),
    mesh=_mesh,
    compiler_params=_sc_params,
    scratch_types=[
        pltpu.VMEM((SIDX_W,), i32),
        pltpu.VMEM((SIDX_W, H), f32),
        pltpu.VMEM((2, H), f32),
        pltpu.SemaphoreType.DMA,
    ],
)
def _sheet_kernel(h_hbm, sidx_hbm, out_hbm, idx_stage, rows, sums, sem):
    c = lax.axis_index("c")
    s = lax.axis_index("s")
    wid = c * NS + s

    pltpu.sync_copy(sidx_hbm.at[pl.ds(wid * SIDX_W, SIDX_W)], idx_stage)
    for k in range(SIDX_W // GCH):
        pltpu.async_copy(h_hbm.at[idx_stage.at[pl.ds(k * GCH, GCH)]],
                         rows.at[pl.ds(k * GCH, GCH)], sem).wait()

    for j in range(2):
        def red(r, carry):
            a0, a1, a2, a3 = carry
            base = j * L + r
            a0 = a0 + rows[base, pl.ds(0, 16)]
            a1 = a1 + rows[base, pl.ds(16, 16)]
            a2 = a2 + rows[base, pl.ds(32, 16)]
            a3 = a3 + rows[base, pl.ds(48, 16)]
            return (a0, a1, a2, a3)

        z = jnp.zeros((16,), f32)
        a0, a1, a2, a3 = lax.fori_loop(0, L, red, (z, z, z, z))
        sums[j, pl.ds(0, 16)] = a0
        sums[j, pl.ds(16, 16)] = a1
        sums[j, pl.ds(32, 16)] = a2
        sums[j, pl.ds(48, 16)] = a3

    pltpu.sync_copy(sums, out_hbm.at[wid])


NPK = N // 2
NPP = NP // 2


def _tc_proj_body(xp_ref, w0b_ref, p_ref):
    p_ref[...] = jnp.dot(xp_ref[...], w0b_ref[...],
                         preferred_element_type=f32)


def _tc_scale_body(p_ref, cntp_ref, ps_ref, dinvb_ref):
    degb = cntp_ref[0, 0:NPK] + cntp_ref[1, 0:NPK] + 1.0
    dinvb = lax.rsqrt(degb)
    dinvb_ref[...] = dinvb
    ps_ref[...] = p_ref[...] * dinvb


def _tc_layer_body(s_ref, ps_ref, dinvb_ref, b_ref, wb_ref, out_ref):
    dinvb = dinvb_ref[...]
    h = jnp.maximum(
        (s_ref[0, 0:NPK] + s_ref[1, 0:NPK] + ps_ref[...]) * dinvb
        + b_ref[...], 0.0)
    out_ref[...] = jnp.dot(h, wb_ref[...], preferred_element_type=f32) * dinvb


def _tc_finalh_body(s_ref, ps_ref, dinvb_ref, b_ref, out_ref):
    out_ref[...] = jnp.maximum(
        (s_ref[0, 0:NPK] + s_ref[1, 0:NPK] + ps_ref[...]) * dinvb_ref[...]
        + b_ref[...], 0.0)


def _tc_gmean_body(h_ref, g_ref):
    g_ref[...] = jnp.sum(h_ref[...], axis=0, keepdims=True) * (1.0 / N)


def _tc_head_body(g2_ref, ss_ref, wq1_ref, bq1_ref, wq2_ref, bq2_ref, q_ref):
    g2 = g2_ref[...]
    g = g2[:, 0:H] + g2[:, H:2 * H]
    se = ss_ref[...] * (1.0 / L)
    hcat = jnp.concatenate([se, jnp.broadcast_to(g, (S, H))], axis=1)
    z = jnp.maximum(
        jnp.dot(hcat, wq1_ref[...], preferred_element_type=f32) + bq1_ref[...],
        0.0)
    q_ref[...] = jnp.dot(z, wq2_ref[...], preferred_element_type=f32) + bq2_ref[...]


_tc_proj = pl.pallas_call(
    _tc_proj_body,
    out_shape=jax.ShapeDtypeStruct((NPK, 2 * H), f32),
)

_tc_scale = pl.pallas_call(
    _tc_scale_body,
    out_shape=[jax.ShapeDtypeStruct((NPK, 2 * H), f32),
               jax.ShapeDtypeStruct((NPK, 2 * H), f32)],
)

_tc_layer = pl.pallas_call(
    _tc_layer_body,
    out_shape=jax.ShapeDtypeStruct((NPK, 2 * H), f32),
)

_tc_finalh = pl.pallas_call(
    _tc_finalh_body,
    out_shape=jax.ShapeDtypeStruct((NPK, 2 * H), f32),
)

_tc_gmean = pl.pallas_call(
    _tc_gmean_body,
    out_shape=jax.ShapeDtypeStruct((1, 2 * H), f32),
)

_tc_head = pl.pallas_call(
    _tc_head_body,
    out_shape=jax.ShapeDtypeStruct((S, 1), f32),
)


def _blockdiag2(w):
    fi, fo = w.shape
    z = jnp.zeros((fi, fo), f32)
    return jnp.concatenate(
        [jnp.concatenate([w, z], axis=1),
         jnp.concatenate([z, w], axis=1)], axis=0)


def kernel(x, edge_index, batch, sheet_node_idx, W0, b0, W1, b1, W2, b2,
           Wq1, bq1, Wq2, bq2):
    del batch
    ei = edge_index.astype(i32)
    src = ei[0]
    dst = ei[1]
    sidx = sheet_node_idx.astype(i32).reshape(-1)
    zeros64 = jnp.zeros((NP, H), f32)
    zeros16 = jnp.zeros((NP, 16), f32)

    def pack2(b2d):
        return jnp.concatenate([b2d, b2d]).reshape(1, 2 * H)

    def unpack_sc(a):
        return a.reshape(N, H)

    def pack_s(sarr):
        return sarr.reshape(NC, NPP, 2 * H)

    cnt = _deg_kernel(dst, zeros16)
    xp = x.reshape(NPK, 2 * D)
    p1 = _tc_proj(xp, _blockdiag2(W0))
    ps1, dinvb = _tc_scale(p1, pack_s(cnt))
    s1 = _mp_kernel(unpack_sc(ps1), src, dst, zeros64)
    ps2 = _tc_layer(pack_s(s1), ps1, dinvb, pack2(b0), _blockdiag2(W1))
    s2 = _mp_kernel(unpack_sc(ps2), src, dst, zeros64)
    ps3 = _tc_layer(pack_s(s2), ps2, dinvb, pack2(b1), _blockdiag2(W2))
    s3 = _mp_kernel(unpack_sc(ps3), src, dst, zeros64)
    h3 = _tc_finalh(pack_s(s3), ps3, dinvb, pack2(b2))
    g2 = _tc_gmean(h3)
    ss = _sheet_kernel(unpack_sc(h3), sidx).reshape(S, H)
    q = _tc_head(g2, ss, Wq1, bq1.reshape(1, H), Wq2, bq2.reshape(1, 1))
    return q.squeeze(-1)


# speedup vs baseline: 1.0507x; 1.0507x over previous
import functools

import jax
import jax.numpy as jnp
from jax import lax
from jax.experimental import pallas as pl
from jax.experimental.pallas import tpu as pltpu
from jax.experimental.pallas import tpu_sc as plsc

N = 10000
E = 320000
D = 128
H = 64
S = 64
L = 256

NC = 2
NS = 16
NW = NC * NS
EW = E // NW
C = 80
NCH = EW // C
CM = 40
NCHM = EW // CM
NP = 10240
RT = NP // NS

_mesh = plsc.VectorSubcoreMesh(core_axis_name="c", subcore_axis_name="s")
_sc_params = pltpu.CompilerParams(use_tc_tiling_on_sc=False)

f32 = jnp.float32
i32 = jnp.int32


@functools.partial(
    pl.kernel,
    out_type=jax.ShapeDtypeStruct((NC, NP, H), f32),
    mesh=_mesh,
    compiler_params=_sc_params,
    scratch_types=[
        pltpu.VMEM((EW,), i32),
        pltpu.VMEM((C, 16), f32),
        pltpu.VMEM((RT, 16), f32),
        pltpu.VMEM((RT, H), f32),
        pltpu.VMEM_SHARED((NP, 16), f32),
        pltpu.SemaphoreType.DMA,
    ],
)
def _deg_kernel(dst_hbm, zeros16_hbm, out_hbm, idx_stage, ones_c,
                cnt16, cnt64, acc, sem):
    c = lax.axis_index("c")
    s = lax.axis_index("s")
    wid = c * NS + s

    pltpu.sync_copy(zeros16_hbm.at[pl.ds(s * RT, RT)],
                    acc.at[pl.ds(s * RT, RT)])

    def fill(i, carry):
        ones_c[i, :] = jnp.full((16,), 1.0, f32)
        return carry

    lax.fori_loop(0, C, fill, 0)
    pltpu.sync_copy(dst_hbm.at[pl.ds(wid * EW, EW)], idx_stage)
    plsc.subcore_barrier()

    def chunk(j, carry):
        pltpu.async_copy(ones_c, acc.at[idx_stage.at[pl.ds(j * C, C)]],
                         sem, add=True)
        return carry

    lax.fori_loop(0, NCH, chunk, 0)

    def drain(j, carry):
        pltpu.make_async_copy(ones_c, acc.at[idx_stage.at[pl.ds(0, C)]],
                              sem).wait()
        return carry

    lax.fori_loop(0, NCH, drain, 0)
    plsc.subcore_barrier()

    pltpu.sync_copy(acc.at[pl.ds(s * RT, RT)], cnt16)

    def expand(r, carry):
        v = cnt16[r, :]
        for k in range(H // 16):
            cnt64[r, pl.ds(16 * k, 16)] = v
        return carry

    lax.fori_loop(0, RT, expand, 0)
    pltpu.sync_copy(cnt64, out_hbm.at[c, pl.ds(s * RT, RT)])


@functools.partial(
    pl.kernel,
    out_type=jax.ShapeDtypeStruct((NC, NP, H), f32),
    mesh=_mesh,
    compiler_params=_sc_params,
    scratch_types=[
        pltpu.VMEM((EW,), i32),
        pltpu.VMEM((EW,), i32),
        pltpu.VMEM((3 * 5, CM, H), f32),
        pltpu.VMEM_SHARED((NP, H), f32),
        pltpu.SemaphoreType.DMA,
        pltpu.SemaphoreType.DMA,
        pltpu.SemaphoreType.DMA,
        pltpu.SemaphoreType.DMA,
        pltpu.SemaphoreType.DMA,
        pltpu.SemaphoreType.DMA,
    ],
)
def _mp_kernel(ps_hbm, src_hbm, dst_hbm, zeros_hbm, out_hbm,
               src_stage, dst_stage, bufs, acc,
               gsem0, gsem1, gsem2, ssem0, ssem1, ssem2):
    c = lax.axis_index("c")
    s = lax.axis_index("s")
    wid = c * NS + s
    gsems = (gsem0, gsem1, gsem2)
    ssems = (ssem0, ssem1, ssem2)
    R = 5
    NG = NCHM // R

    pltpu.sync_copy(zeros_hbm.at[pl.ds(s * RT, RT)], acc.at[pl.ds(s * RT, RT)])
    pltpu.sync_copy(src_hbm.at[pl.ds(wid * EW, EW)], src_stage)
    pltpu.sync_copy(dst_hbm.at[pl.ds(wid * EW, EW)], dst_stage)
    plsc.subcore_barrier()

    def gather(j, p, b):
        pltpu.async_copy(ps_hbm.at[src_stage.at[pl.ds(j * CM, CM)]],
                         bufs.at[p * R + b], gsems[p])

    def drain_scatters(p):
        for b in range(R):
            pltpu.make_async_copy(bufs.at[p * R + b],
                                  acc.at[dst_stage.at[pl.ds(0, CM)]],
                                  ssems[p]).wait()

    def slot(g, p, drain_prev=True, refill=True):
        for b in range(R):
            pltpu.make_async_copy(ps_hbm.at[pl.ds(0, CM)],
                                  bufs.at[p * R + b], gsems[p]).wait()
        for b in range(R):
            j = g * R + b
            pltpu.async_copy(bufs.at[p * R + b],
                             acc.at[dst_stage.at[pl.ds(j * CM, CM)]],
                             ssems[p], add=True)
        p1 = (p + 2) % 3
        if drain_prev:
            @pl.when(g > 0)
            def _():
                drain_scatters(p1)
        if refill:
            @pl.when(g + 2 < NG)
            def _():
                for b in range(R):
                    gather((g + 2) * R + b, p1, b)

    for p in (0, 1):
        for b in range(R):
            gather(p * R + b, p, b)

    def outer(gg, carry):
        for p in (0, 1, 2):
            slot(3 * gg + p, p)
        return carry

    lax.fori_loop(0, NG // 3, outer, 0)
    slot(NG - 2, (NG - 2) % 3, refill=False)
    slot(NG - 1, (NG - 1) % 3, refill=False)
    drain_scatters((NG - 1) % 3)

    plsc.subcore_barrier()
    pltpu.sync_copy(acc.at[pl.ds(s * RT, RT)], out_hbm.at[c, pl.ds(s * RT, RT)])


SIDX_W = S * L // NW
GCH = 128


@functools.partial(
    pl.kernel,
    out_type=jax.ShapeDtypeStruct((NW, 2, H), f32),
    mesh=_mesh,
    compiler_params=_sc_params,
    scratch_types=[
        pltpu.VMEM((SIDX_W,), i32),
        pltpu.VMEM((SIDX_W, H), f32),
        pltpu.VMEM((2, H), f32),
        pltpu.SemaphoreType.DMA,
    ],
)
def _sheet_kernel(h_hbm, sidx_hbm, out_hbm, idx_stage, rows, sums, sem):
    c = lax.axis_index("c")
    s = lax.axis_index("s")
    wid = c * NS + s

    pltpu.sync_copy(sidx_hbm.at[pl.ds(wid * SIDX_W, SIDX_W)], idx_stage)
    for k in range(SIDX_W // GCH):
        pltpu.async_copy(h_hbm.at[idx_stage.at[pl.ds(k * GCH, GCH)]],
                         rows.at[pl.ds(k * GCH, GCH)], sem).wait()

    for j in range(2):
        def red(r, carry):
            a0, a1, a2, a3 = carry
            base = j * L + r
            a0 = a0 + rows[base, pl.ds(0, 16)]
            a1 = a1 + rows[base, pl.ds(16, 16)]
            a2 = a2 + rows[base, pl.ds(32, 16)]
            a3 = a3 + rows[base, pl.ds(48, 16)]
            return (a0, a1, a2, a3)

        z = jnp.zeros((16,), f32)
        a0, a1, a2, a3 = lax.fori_loop(0, L, red, (z, z, z, z))
        sums[j, pl.ds(0, 16)] = a0
        sums[j, pl.ds(16, 16)] = a1
        sums[j, pl.ds(32, 16)] = a2
        sums[j, pl.ds(48, 16)] = a3

    pltpu.sync_copy(sums, out_hbm.at[wid])


NPK = N // 2
NPP = NP // 2


def _tc_proj_body(xp_ref, w0b_ref, p_ref):
    p_ref[...] = jnp.dot(xp_ref[...], w0b_ref[...],
                         preferred_element_type=f32)


def _tc_scale_body(p_ref, cntp_ref, ps_ref, dinvb_ref):
    degb = cntp_ref[0, 0:NPK] + cntp_ref[1, 0:NPK] + 1.0
    dinvb = lax.rsqrt(degb)
    dinvb_ref[...] = dinvb
    ps_ref[...] = p_ref[...] * dinvb


def _tc_layer_body(s_ref, ps_ref, dinvb_ref, b_ref, wb_ref, out_ref):
    dinvb = dinvb_ref[...]
    h = jnp.maximum(
        (s_ref[0, 0:NPK] + s_ref[1, 0:NPK] + ps_ref[...]) * dinvb
        + b_ref[...], 0.0)
    out_ref[...] = jnp.dot(h, wb_ref[...], preferred_element_type=f32) * dinvb


def _tc_finalh_body(s_ref, ps_ref, dinvb_ref, b_ref, out_ref):
    out_ref[...] = jnp.maximum(
        (s_ref[0, 0:NPK] + s_ref[1, 0:NPK] + ps_ref[...]) * dinvb_ref[...]
        + b_ref[...], 0.0)


def _tc_gmean_body(h_ref, g_ref):
    g_ref[...] = jnp.sum(h_ref[...], axis=0, keepdims=True) * (1.0 / N)


def _tc_head_body(g2_ref, ss_ref, wq1_ref, bq1_ref, wq2_ref, bq2_ref, q_ref):
    g2 = g2_ref[...]
    g = g2[:, 0:H] + g2[:, H:2 * H]
    se = ss_ref[...] * (1.0 / L)
    hcat = jnp.concatenate([se, jnp.broadcast_to(g, (S, H))], axis=1)
    z = jnp.maximum(
        jnp.dot(hcat, wq1_ref[...], preferred_element_type=f32) + bq1_ref[...],
        0.0)
    q_ref[...] = jnp.dot(z, wq2_ref[...], preferred_element_type=f32) + bq2_ref[...]


_tc_proj = pl.pallas_call(
    _tc_proj_body,
    out_shape=jax.ShapeDtypeStruct((NPK, 2 * H), f32),
)

_tc_scale = pl.pallas_call(
    _tc_scale_body,
    out_shape=[jax.ShapeDtypeStruct((NPK, 2 * H), f32),
               jax.ShapeDtypeStruct((NPK, 2 * H), f32)],
)

_tc_layer = pl.pallas_call(
    _tc_layer_body,
    out_shape=jax.ShapeDtypeStruct((NPK, 2 * H), f32),
)

_tc_finalh = pl.pallas_call(
    _tc_finalh_body,
    out_shape=jax.ShapeDtypeStruct((NPK, 2 * H), f32),
)

_tc_gmean = pl.pallas_call(
    _tc_gmean_body,
    out_shape=jax.ShapeDtypeStruct((1, 2 * H), f32),
)

_tc_head = pl.pallas_call(
    _tc_head_body,
    out_shape=jax.ShapeDtypeStruct((S, 1), f32),
)


def _blockdiag2(w):
    fi, fo = w.shape
    z = jnp.zeros((fi, fo), f32)
    return jnp.concatenate(
        [jnp.concatenate([w, z], axis=1),
         jnp.concatenate([z, w], axis=1)], axis=0)


def kernel(x, edge_index, batch, sheet_node_idx, W0, b0, W1, b1, W2, b2,
           Wq1, bq1, Wq2, bq2):
    del batch
    ei = edge_index.astype(i32)
    src = ei[0]
    dst = ei[1]
    sidx = sheet_node_idx.astype(i32).reshape(-1)
    zeros64 = jnp.zeros((NP, H), f32)
    zeros16 = jnp.zeros((NP, 16), f32)

    def pack2(b2d):
        return jnp.concatenate([b2d, b2d]).reshape(1, 2 * H)

    def unpack_sc(a):
        return a.reshape(N, H)

    def pack_s(sarr):
        return sarr.reshape(NC, NPP, 2 * H)

    cnt = _deg_kernel(dst, zeros16)
    xp = x.reshape(NPK, 2 * D)
    p1 = _tc_proj(xp, _blockdiag2(W0))
    ps1, dinvb = _tc_scale(p1, pack_s(cnt))
    s1 = _mp_kernel(unpack_sc(ps1), src, dst, zeros64)
    ps2 = _tc_layer(pack_s(s1), ps1, dinvb, pack2(b0), _blockdiag2(W1))
    s2 = _mp_kernel(unpack_sc(ps2), src, dst, zeros64)
    ps3 = _tc_layer(pack_s(s2), ps2, dinvb, pack2(b1), _blockdiag2(W2))
    s3 = _mp_kernel(unpack_sc(ps3), src, dst, zeros64)
    h3 = _tc_finalh(pack_s(s3), ps3, dinvb, pack2(b2))
    g2 = _tc_gmean(h3)
    ss = _sheet_kernel(unpack_sc(h3), sidx).reshape(S, H)
    q = _tc_head(g2, ss, Wq1, bq1.reshape(1, H), Wq2, bq2.reshape(1, 1))
    return q.squeeze(-1)
